# baseline (speedup 1.0000x reference)
import functools

import jax
import jax.numpy as jnp
from jax import lax
from jax.experimental import pallas as pl
from jax.experimental.pallas import tpu as pltpu
from jax.experimental.pallas import tpu_sc as plsc

_N = 4096
_S = 512
_TS = 256
_CIN = 67
_D = 128
_K = 48
_RADII = (0.1, 0.2, 0.4)
_NS = (16, 32, 48)
_COUT = 128
_NW = 32
_CHS = 512
_TS1 = 256


def _sel_kernel(xyzT_ref, cen_ref, idx_ref, d2_s, ix_s):
    cx = cen_ref[0, :, 0:1]
    cy = cen_ref[0, :, 1:2]
    cz = cen_ref[0, :, 2:3]
    px = xyzT_ref[0, 0:1, :]
    py = xyzT_ref[0, 1:2, :]
    pz = xyzT_ref[0, 2:3, :]
    dx = cx - px
    dy = cy - py
    dz = cz - pz
    d2_s[...] = dx * dx + dy * dy + dz * dz

    ii = jax.lax.broadcasted_iota(jnp.int32, (_TS1, _N), 1)
    base = pl.program_id(0) * _N

    def step(t, carry):
        mp, ip = carry
        d2 = d2_s[...]
        pred = (d2 > mp) | ((d2 == mp) & (ii > ip))
        dm = jnp.where(pred, d2, jnp.float32(1e30))
        m = jnp.min(dm, axis=1, keepdims=True)
        cand = jnp.where(dm == m, ii, jnp.int32(_N))
        im = jnp.min(cand, axis=1, keepdims=True)
        ix_s[t] = im + base
        return m, im

    jax.lax.fori_loop(
        0, _K, step,
        (jnp.full((_TS1, 1), -1.0, jnp.float32),
         jnp.zeros((_TS1, 1), jnp.int32)))
    idx_ref[0] = ix_s[...]


def _make_gather(m_total):
    per_w = m_total // _NW
    n_chunks = per_w // _CHS
    mesh = plsc.VectorSubcoreMesh(core_axis_name="c", subcore_axis_name="s")

    @functools.partial(
        pl.kernel, mesh=mesh,
        out_type=jax.ShapeDtypeStruct((m_total, _D), jnp.float32),
        scratch_types=[
            pltpu.VMEM((_CHS,), jnp.int32),
            pltpu.VMEM((_CHS, _D), jnp.float32),
            pltpu.SemaphoreType.DMA,
        ],
    )
    def gk(table_hbm, idx_hbm, out_hbm, idx_v, rows_v, sem):
        wid = lax.axis_index("s") * 2 + lax.axis_index("c")
        base = wid * per_w

        def chunk(i, carry):
            off = base + i * _CHS
            pltpu.sync_copy(idx_hbm.at[pl.ds(off, _CHS)], idx_v)
            pltpu.async_copy(table_hbm.at[idx_v], rows_v, sem).wait()
            pltpu.sync_copy(rows_v, out_hbm.at[pl.ds(off, _CHS)])
            return carry

        lax.fori_loop(0, n_chunks, chunk, 0)

    return gk


def _mlp_kernel(g_ref, cen_ref, wn_ref, wc_ref, wp_ref, b1_ref,
                m2_ref, b2_ref, brs_ref, crw_ref, crb_ref, out_ref):
    cen = cen_ref[0]
    ccx = cen[:, 0:1]
    ccy = cen[:, 1:2]
    ccz = cen[:, 2:3]
    cc = jnp.dot(cen, wc_ref[...], preferred_element_type=jnp.float32,
                 precision=jax.lax.Precision.HIGHEST)

    _CH = 16
    ymax = [None, None, None]
    for c0 in range(0, _K, _CH):
        g = g_ref[0, c0:c0 + _CH]
        p3 = g[:, :, 0:3]
        feats = g[:, :, 3:_CIN]
        dxx = ccx[None] - p3[:, :, 0:1]
        dyy = ccy[None] - p3[:, :, 1:2]
        dzz = ccz[None] - p3[:, :, 2:3]
        mv = dxx * dxx + dyy * dyy + dzz * dzz
        nrm = jnp.sqrt(mv + 1e-12)
        pp = (p3[:, :, 0:1] * wp_ref[0:1, :][None]
              + p3[:, :, 1:2] * wp_ref[1:2, :][None]
              + p3[:, :, 2:3] * wp_ref[2:3, :][None])
        u = nrm * wn_ref[...][None] + cc[None] + pp + b1_ref[...][None]
        u = jnp.maximum(u, 0.0)
        h2 = jnp.dot(u.reshape(_CH * _TS, 32), m2_ref[...],
                     preferred_element_type=jnp.float32,
                     precision=jax.lax.Precision.HIGHEST)
        h2 = h2.reshape(_CH, _TS, _CIN) + b2_ref[...][None]
        delta = jnp.concatenate([-dxx, -dyy, -dzz], axis=-1)
        xcat = jnp.concatenate([delta, feats], axis=-1)
        z = h2 * xcat
        tio = jax.lax.broadcasted_iota(jnp.int32, (_CH, 1, 1), 0) + c0
        for r_i in range(3):
            rad = _RADII[r_i]
            ns = _NS[r_i]
            if c0 >= ns:
                continue
            mask = (mv <= jnp.float32(rad * rad)) & (tio < ns)
            y = jnp.max(jnp.where(mask, z, jnp.float32(-1e30)), axis=0)
            ymax[r_i] = y if ymax[r_i] is None else jnp.maximum(ymax[r_i], y)

    for r_i in range(3):
        y = jnp.maximum(ymax[r_i] + brs_ref[...], 0.0)
        o = jnp.dot(y, crw_ref[...], preferred_element_type=jnp.float32,
                    precision=jax.lax.Precision.HIGHEST)
        o = jnp.maximum(o + crb_ref[...], 0.0)
        out_ref[0, :, r_i * _COUT:(r_i + 1) * _COUT] = o


def kernel(xyz, features, m1_w, m1_b, m2_w, m2_b, cr_w, cr_b,
           g_map, b_map, g_rs, b_rs, g_cr, b_cr):
    B = xyz.shape[0]
    featT = jnp.transpose(features, (0, 2, 1))
    pf = jnp.concatenate(
        [xyz, featT, jnp.zeros((B, _N, _D - _CIN), jnp.float32)], axis=-1)
    table = pf.reshape(B * _N, _D)
    xyzT = jnp.transpose(xyz, (0, 2, 1))
    cen = xyz[:, :_S, :]

    idx = pl.pallas_call(
        _sel_kernel,
        grid=(B, _S // _TS1),
        in_specs=[
            pl.BlockSpec((1, 3, _N), lambda b, s: (b, 0, 0)),
            pl.BlockSpec((1, _TS1, 3), lambda b, s: (b, s, 0)),
        ],
        out_specs=pl.BlockSpec((1, _K, _TS1, 1), lambda b, s: (b, 0, s, 0)),
        out_shape=jax.ShapeDtypeStruct((B, _K, _S, 1), jnp.int32),
        scratch_shapes=[
            pltpu.VMEM((_TS1, _N), jnp.float32),
            pltpu.VMEM((_K, _TS1, 1), jnp.int32),
        ],
    )(xyzT, cen)

    m_total = B * _K * _S
    g = _make_gather(m_total)(table, idx.reshape(m_total))
    g = g.reshape(B, _K, _S, _D)

    w1 = g_map[:, None] * m1_w
    b1 = (g_map * m1_b + b_map)[None, :]
    wn = w1[:, 0:1].T
    wc = (w1[:, 1:4] - w1[:, 7:10]).T
    wp = (w1[:, 4:7] + w1[:, 7:10]).T
    m2f = (g_rs[:, None] * m2_w).T
    b2f = (g_rs * m2_b)[None, :]
    brs = b_rs[None, :]
    crw = (g_cr[:, None] * cr_w).T
    crb = (g_cr * cr_b + b_cr)[None, :]

    out = pl.pallas_call(
        _mlp_kernel,
        grid=(B, _S // _TS),
        in_specs=[
            pl.BlockSpec((1, _K, _TS, _D), lambda b, s: (b, 0, s, 0)),
            pl.BlockSpec((1, _TS, 3), lambda b, s: (b, s, 0)),
            pl.BlockSpec((1, 32), lambda b, s: (0, 0)),
            pl.BlockSpec((3, 32), lambda b, s: (0, 0)),
            pl.BlockSpec((3, 32), lambda b, s: (0, 0)),
            pl.BlockSpec((1, 32), lambda b, s: (0, 0)),
            pl.BlockSpec((32, _CIN), lambda b, s: (0, 0)),
            pl.BlockSpec((1, _CIN), lambda b, s: (0, 0)),
            pl.BlockSpec((1, _CIN), lambda b, s: (0, 0)),
            pl.BlockSpec((_CIN, _COUT), lambda b, s: (0, 0)),
            pl.BlockSpec((1, _COUT), lambda b, s: (0, 0)),
        ],
        out_specs=pl.BlockSpec((1, _TS, 3 * _COUT), lambda b, s: (b, s, 0)),
        out_shape=jax.ShapeDtypeStruct((B, _S, 3 * _COUT), jnp.float32),
    )(g, cen, wn, wc, wp, b1, m2f, b2f, brs, crw, crb)
    return cen, jnp.transpose(out, (0, 2, 1))


# speedup vs baseline: 1.3946x; 1.3946x over previous
import functools

import jax
import jax.numpy as jnp
from jax import lax
from jax.experimental import pallas as pl
from jax.experimental.pallas import tpu as pltpu
from jax.experimental.pallas import tpu_sc as plsc

_N = 4096
_S = 512
_TS = 256
_CIN = 67
_D = 128
_K = 48
_RADII = (0.1, 0.2, 0.4)
_NS = (16, 32, 48)
_COUT = 128
_NW = 32
_CHS = 512
_TS1 = 256


def _sel_kernel(xyzT_ref, cen_ref, idx_ref, d2_s, ix_s):
    cx = cen_ref[0, :, 0:1]
    cy = cen_ref[0, :, 1:2]
    cz = cen_ref[0, :, 2:3]
    px = xyzT_ref[0, 0:1, :]
    py = xyzT_ref[0, 1:2, :]
    pz = xyzT_ref[0, 2:3, :]
    dx = cx - px
    dy = cy - py
    dz = cz - pz
    d2_s[...] = dx * dx + dy * dy + dz * dz

    ii = jax.lax.broadcasted_iota(jnp.int32, (_TS1, _N), 1)
    base = pl.program_id(0) * _N

    _U = 4

    def step(t, carry):
        d2 = d2_s[...]
        for u in range(_U):
            m = jnp.min(d2, axis=1, keepdims=True)
            cand = jnp.where(d2 == m, ii, jnp.int32(_N))
            im = jnp.min(cand, axis=1, keepdims=True)
            ix_s[t * _U + u] = im + base
            d2 = jnp.where(ii == im, jnp.float32(1e30), d2)
        d2_s[...] = d2
        return carry

    jax.lax.fori_loop(0, _K // _U, step, 0)
    idx_ref[0] = ix_s[...]


def _make_gather(m_total):
    per_w = m_total // _NW
    n_chunks = per_w // _CHS
    mesh = plsc.VectorSubcoreMesh(core_axis_name="c", subcore_axis_name="s")

    @functools.partial(
        pl.kernel, mesh=mesh,
        out_type=jax.ShapeDtypeStruct((m_total, _D), jnp.float32),
        scratch_types=[
            pltpu.VMEM((_CHS,), jnp.int32),
            pltpu.VMEM((_CHS, _D), jnp.float32),
            pltpu.SemaphoreType.DMA,
        ],
    )
    def gk(table_hbm, idx_hbm, out_hbm, idx_v, rows_v, sem):
        wid = lax.axis_index("s") * 2 + lax.axis_index("c")
        base = wid * per_w

        def chunk(i, carry):
            off = base + i * _CHS
            pltpu.sync_copy(idx_hbm.at[pl.ds(off, _CHS)], idx_v)
            pltpu.async_copy(table_hbm.at[idx_v], rows_v, sem).wait()
            pltpu.sync_copy(rows_v, out_hbm.at[pl.ds(off, _CHS)])
            return carry

        lax.fori_loop(0, n_chunks, chunk, 0)

    return gk


def _mlp_kernel(g_ref, cen_ref, wn_ref, wc_ref, wp_ref, b1_ref,
                m2_ref, b2_ref, brs_ref, crw_ref, crb_ref, out_ref):
    cen = cen_ref[0]
    ccx = cen[:, 0:1]
    ccy = cen[:, 1:2]
    ccz = cen[:, 2:3]
    cc = jnp.dot(cen, wc_ref[...], preferred_element_type=jnp.float32,
                 precision=jax.lax.Precision.HIGHEST)

    _CH = 16
    ymax = [None, None, None]
    for c0 in range(0, _K, _CH):
        g = g_ref[0, c0:c0 + _CH]
        p3 = g[:, :, 0:3]
        feats = g[:, :, 3:_CIN]
        dxx = ccx[None] - p3[:, :, 0:1]
        dyy = ccy[None] - p3[:, :, 1:2]
        dzz = ccz[None] - p3[:, :, 2:3]
        mv = dxx * dxx + dyy * dyy + dzz * dzz
        nrm = jnp.sqrt(mv + 1e-12)
        pp = (p3[:, :, 0:1] * wp_ref[0:1, :][None]
              + p3[:, :, 1:2] * wp_ref[1:2, :][None]
              + p3[:, :, 2:3] * wp_ref[2:3, :][None])
        u = nrm * wn_ref[...][None] + cc[None] + pp + b1_ref[...][None]
        u = jnp.maximum(u, 0.0)
        h2 = jnp.dot(u.reshape(_CH * _TS, 32), m2_ref[...],
                     preferred_element_type=jnp.float32,
                     precision=jax.lax.Precision.HIGHEST)
        h2 = h2.reshape(_CH, _TS, _CIN) + b2_ref[...][None]
        delta = jnp.concatenate([-dxx, -dyy, -dzz], axis=-1)
        xcat = jnp.concatenate([delta, feats], axis=-1)
        z = h2 * xcat
        tio = jax.lax.broadcasted_iota(jnp.int32, (_CH, 1, 1), 0) + c0
        for r_i in range(3):
            rad = _RADII[r_i]
            ns = _NS[r_i]
            if c0 >= ns:
                continue
            mask = (mv <= jnp.float32(rad * rad)) & (tio < ns)
            y = jnp.max(jnp.where(mask, z, jnp.float32(-1e30)), axis=0)
            ymax[r_i] = y if ymax[r_i] is None else jnp.maximum(ymax[r_i], y)

    for r_i in range(3):
        y = jnp.maximum(ymax[r_i] + brs_ref[...], 0.0)
        o = jnp.dot(y, crw_ref[...], preferred_element_type=jnp.float32,
                    precision=jax.lax.Precision.HIGHEST)
        o = jnp.maximum(o + crb_ref[...], 0.0)
        out_ref[0, :, r_i * _COUT:(r_i + 1) * _COUT] = o


def kernel(xyz, features, m1_w, m1_b, m2_w, m2_b, cr_w, cr_b,
           g_map, b_map, g_rs, b_rs, g_cr, b_cr):
    B = xyz.shape[0]
    featT = jnp.transpose(features, (0, 2, 1))
    pf = jnp.concatenate(
        [xyz, featT, jnp.zeros((B, _N, _D - _CIN), jnp.float32)], axis=-1)
    table = pf.reshape(B * _N, _D)
    xyzT = jnp.transpose(xyz, (0, 2, 1))
    cen = xyz[:, :_S, :]

    idx = pl.pallas_call(
        _sel_kernel,
        grid=(B, _S // _TS1),
        in_specs=[
            pl.BlockSpec((1, 3, _N), lambda b, s: (b, 0, 0)),
            pl.BlockSpec((1, _TS1, 3), lambda b, s: (b, s, 0)),
        ],
        out_specs=pl.BlockSpec((1, _K, _TS1, 1), lambda b, s: (b, 0, s, 0)),
        out_shape=jax.ShapeDtypeStruct((B, _K, _S, 1), jnp.int32),
        scratch_shapes=[
            pltpu.VMEM((_TS1, _N), jnp.float32),
            pltpu.VMEM((_K, _TS1, 1), jnp.int32),
        ],
    )(xyzT, cen)

    m_total = B * _K * _S
    g = _make_gather(m_total)(table, idx.reshape(m_total))
    g = g.reshape(B, _K, _S, _D)

    w1 = g_map[:, None] * m1_w
    b1 = (g_map * m1_b + b_map)[None, :]
    wn = w1[:, 0:1].T
    wc = (w1[:, 1:4] - w1[:, 7:10]).T
    wp = (w1[:, 4:7] + w1[:, 7:10]).T
    m2f = (g_rs[:, None] * m2_w).T
    b2f = (g_rs * m2_b)[None, :]
    brs = b_rs[None, :]
    crw = (g_cr[:, None] * cr_w).T
    crb = (g_cr * cr_b + b_cr)[None, :]

    out = pl.pallas_call(
        _mlp_kernel,
        grid=(B, _S // _TS),
        in_specs=[
            pl.BlockSpec((1, _K, _TS, _D), lambda b, s: (b, 0, s, 0)),
            pl.BlockSpec((1, _TS, 3), lambda b, s: (b, s, 0)),
            pl.BlockSpec((1, 32), lambda b, s: (0, 0)),
            pl.BlockSpec((3, 32), lambda b, s: (0, 0)),
            pl.BlockSpec((3, 32), lambda b, s: (0, 0)),
            pl.BlockSpec((1, 32), lambda b, s: (0, 0)),
            pl.BlockSpec((32, _CIN), lambda b, s: (0, 0)),
            pl.BlockSpec((1, _CIN), lambda b, s: (0, 0)),
            pl.BlockSpec((1, _CIN), lambda b, s: (0, 0)),
            pl.BlockSpec((_CIN, _COUT), lambda b, s: (0, 0)),
            pl.BlockSpec((1, _COUT), lambda b, s: (0, 0)),
        ],
        out_specs=pl.BlockSpec((1, _TS, 3 * _COUT), lambda b, s: (b, s, 0)),
        out_shape=jax.ShapeDtypeStruct((B, _S, 3 * _COUT), jnp.float32),
    )(g, cen, wn, wc, wp, b1, m2f, b2f, brs, crw, crb)
    return cen, jnp.transpose(out, (0, 2, 1))


# speedup vs baseline: 1.4059x; 1.0082x over previous
import functools

import jax
import jax.numpy as jnp
from jax import lax
from jax.experimental import pallas as pl
from jax.experimental.pallas import tpu as pltpu
from jax.experimental.pallas import tpu_sc as plsc

_N = 4096
_S = 512
_TS = 256
_CIN = 67
_D = 128
_K = 48
_RADII = (0.1, 0.2, 0.4)
_NS = (16, 32, 48)
_COUT = 128
_NW = 32
_CHS = 512
_TS1 = 256


def _sel_kernel(xyzT_ref, cen_ref, idx_ref, d2_s, ix_s):
    cx = cen_ref[0, :, 0:1]
    cy = cen_ref[0, :, 1:2]
    cz = cen_ref[0, :, 2:3]
    px = xyzT_ref[0, 0:1, :]
    py = xyzT_ref[0, 1:2, :]
    pz = xyzT_ref[0, 2:3, :]
    dx = cx - px
    dy = cy - py
    dz = cz - pz
    d2_s[...] = dx * dx + dy * dy + dz * dz

    ii = jax.lax.broadcasted_iota(jnp.int32, (_TS1, _N), 1)
    base = pl.program_id(0) * _N

    _U = 8

    def step(t, carry):
        d2 = d2_s[...]
        for u in range(_U):
            m = jnp.min(d2, axis=1, keepdims=True)
            cand = jnp.where(d2 == m, ii, jnp.int32(_N))
            im = jnp.min(cand, axis=1, keepdims=True)
            ix_s[t * _U + u] = im + base
            d2 = jnp.where(ii == im, jnp.float32(1e30), d2)
        d2_s[...] = d2
        return carry

    jax.lax.fori_loop(0, _K // _U, step, 0)
    idx_ref[0] = ix_s[...]


def _make_gather(m_total):
    per_w = m_total // _NW
    n_chunks = per_w // _CHS
    mesh = plsc.VectorSubcoreMesh(core_axis_name="c", subcore_axis_name="s")

    @functools.partial(
        pl.kernel, mesh=mesh,
        out_type=jax.ShapeDtypeStruct((m_total, _D), jnp.float32),
        scratch_types=[
            pltpu.VMEM((_CHS,), jnp.int32),
            pltpu.VMEM((_CHS, _D), jnp.float32),
            pltpu.SemaphoreType.DMA,
        ],
    )
    def gk(table_hbm, idx_hbm, out_hbm, idx_v, rows_v, sem):
        wid = lax.axis_index("s") * 2 + lax.axis_index("c")
        base = wid * per_w

        def chunk(i, carry):
            off = base + i * _CHS
            pltpu.sync_copy(idx_hbm.at[pl.ds(off, _CHS)], idx_v)
            pltpu.async_copy(table_hbm.at[idx_v], rows_v, sem).wait()
            pltpu.sync_copy(rows_v, out_hbm.at[pl.ds(off, _CHS)])
            return carry

        lax.fori_loop(0, n_chunks, chunk, 0)

    return gk


def _mlp_kernel(g_ref, cen_ref, wn_ref, wc_ref, wp_ref, b1_ref,
                m2_ref, b2_ref, brs_ref, crw_ref, crb_ref, out_ref):
    cen = cen_ref[0]
    ccx = cen[:, 0:1]
    ccy = cen[:, 1:2]
    ccz = cen[:, 2:3]
    cc = jnp.dot(cen, wc_ref[...], preferred_element_type=jnp.float32,
                 precision=jax.lax.Precision.HIGHEST)

    _CH = 16
    ymax = [None, None, None]
    for c0 in range(0, _K, _CH):
        g = g_ref[0, c0:c0 + _CH]
        p3 = g[:, :, 0:3]
        feats = g[:, :, 3:_CIN]
        dxx = ccx[None] - p3[:, :, 0:1]
        dyy = ccy[None] - p3[:, :, 1:2]
        dzz = ccz[None] - p3[:, :, 2:3]
        mv = dxx * dxx + dyy * dyy + dzz * dzz
        nrm = jnp.sqrt(mv + 1e-12)
        pp = (p3[:, :, 0:1] * wp_ref[0:1, :][None]
              + p3[:, :, 1:2] * wp_ref[1:2, :][None]
              + p3[:, :, 2:3] * wp_ref[2:3, :][None])
        u = nrm * wn_ref[...][None] + cc[None] + pp + b1_ref[...][None]
        u = jnp.maximum(u, 0.0)
        h2 = jnp.dot(u.reshape(_CH * _TS, 32), m2_ref[...],
                     preferred_element_type=jnp.float32,
                     precision=jax.lax.Precision.HIGHEST)
        h2 = h2.reshape(_CH, _TS, _CIN) + b2_ref[...][None]
        delta = jnp.concatenate([-dxx, -dyy, -dzz], axis=-1)
        xcat = jnp.concatenate([delta, feats], axis=-1)
        z = h2 * xcat
        tio = jax.lax.broadcasted_iota(jnp.int32, (_CH, 1, 1), 0) + c0
        for r_i in range(3):
            rad = _RADII[r_i]
            ns = _NS[r_i]
            if c0 >= ns:
                continue
            mask = (mv <= jnp.float32(rad * rad)) & (tio < ns)
            y = jnp.max(jnp.where(mask, z, jnp.float32(-1e30)), axis=0)
            ymax[r_i] = y if ymax[r_i] is None else jnp.maximum(ymax[r_i], y)

    for r_i in range(3):
        y = jnp.maximum(ymax[r_i] + brs_ref[...], 0.0)
        o = jnp.dot(y, crw_ref[...], preferred_element_type=jnp.float32,
                    precision=jax.lax.Precision.HIGHEST)
        o = jnp.maximum(o + crb_ref[...], 0.0)
        out_ref[0, :, r_i * _COUT:(r_i + 1) * _COUT] = o


def kernel(xyz, features, m1_w, m1_b, m2_w, m2_b, cr_w, cr_b,
           g_map, b_map, g_rs, b_rs, g_cr, b_cr):
    B = xyz.shape[0]
    featT = jnp.transpose(features, (0, 2, 1))
    pf = jnp.concatenate(
        [xyz, featT, jnp.zeros((B, _N, _D - _CIN), jnp.float32)], axis=-1)
    table = pf.reshape(B * _N, _D)
    xyzT = jnp.transpose(xyz, (0, 2, 1))
    cen = xyz[:, :_S, :]

    idx = pl.pallas_call(
        _sel_kernel,
        grid=(B, _S // _TS1),
        in_specs=[
            pl.BlockSpec((1, 3, _N), lambda b, s: (b, 0, 0)),
            pl.BlockSpec((1, _TS1, 3), lambda b, s: (b, s, 0)),
        ],
        out_specs=pl.BlockSpec((1, _K, _TS1, 1), lambda b, s: (b, 0, s, 0)),
        out_shape=jax.ShapeDtypeStruct((B, _K, _S, 1), jnp.int32),
        scratch_shapes=[
            pltpu.VMEM((_TS1, _N), jnp.float32),
            pltpu.VMEM((_K, _TS1, 1), jnp.int32),
        ],
    )(xyzT, cen)

    m_total = B * _K * _S
    g = _make_gather(m_total)(table, idx.reshape(m_total))
    g = g.reshape(B, _K, _S, _D)

    w1 = g_map[:, None] * m1_w
    b1 = (g_map * m1_b + b_map)[None, :]
    wn = w1[:, 0:1].T
    wc = (w1[:, 1:4] - w1[:, 7:10]).T
    wp = (w1[:, 4:7] + w1[:, 7:10]).T
    m2f = (g_rs[:, None] * m2_w).T
    b2f = (g_rs * m2_b)[None, :]
    brs = b_rs[None, :]
    crw = (g_cr[:, None] * cr_w).T
    crb = (g_cr * cr_b + b_cr)[None, :]

    out = pl.pallas_call(
        _mlp_kernel,
        grid=(B, _S // _TS),
        in_specs=[
            pl.BlockSpec((1, _K, _TS, _D), lambda b, s: (b, 0, s, 0)),
            pl.BlockSpec((1, _TS, 3), lambda b, s: (b, s, 0)),
            pl.BlockSpec((1, 32), lambda b, s: (0, 0)),
            pl.BlockSpec((3, 32), lambda b, s: (0, 0)),
            pl.BlockSpec((3, 32), lambda b, s: (0, 0)),
            pl.BlockSpec((1, 32), lambda b, s: (0, 0)),
            pl.BlockSpec((32, _CIN), lambda b, s: (0, 0)),
            pl.BlockSpec((1, _CIN), lambda b, s: (0, 0)),
            pl.BlockSpec((1, _CIN), lambda b, s: (0, 0)),
            pl.BlockSpec((_CIN, _COUT), lambda b, s: (0, 0)),
            pl.BlockSpec((1, _COUT), lambda b, s: (0, 0)),
        ],
        out_specs=pl.BlockSpec((1, _TS, 3 * _COUT), lambda b, s: (b, s, 0)),
        out_shape=jax.ShapeDtypeStruct((B, _S, 3 * _COUT), jnp.float32),
    )(g, cen, wn, wc, wp, b1, m2f, b2f, brs, crw, crb)
    return cen, jnp.transpose(out, (0, 2, 1))
